# baseline (speedup 1.0000x reference)
import functools

import jax
import jax.numpy as jnp
from jax import lax
from jax.experimental import pallas as pl
from jax.experimental.pallas import tpu as pltpu
from jax.experimental.pallas import tpu_sc as plsc

B, T, C = 16, 1024, 32
V = 8192
N_TOK = B * T
TM = 4096
TN = 1024
GRID = N_TOK // TM
NCHUNK = V // TN


def _argmin_body(x_ref, bm_ref, idx_ref):
    xb = x_ref[...]
    x_sq = jnp.sum(xb * xb, axis=1, keepdims=True)
    x_sqb = jnp.broadcast_to(x_sq, (TM, 128))
    runbest = jnp.full((TM, 128), jnp.inf, dtype=jnp.float32)
    runstrip = jnp.zeros((TM, 128), dtype=jnp.int32)
    for j in range(NCHUNK):
        bmj = bm_ref[:, j * TN:(j + 1) * TN]
        y_sq = jnp.sum(bmj * bmj, axis=0) * 0.25
        dotm = lax.dot_general(
            xb, bmj, (((1,), (0,)), ((), ())),
            preferred_element_type=jnp.float32)
        for s in range(0, TN // 128, 4):
            g = j * (TN // 128) + s
            dd = []
            for q in range(4):
                sl = slice((s + q) * 128, (s + q + 1) * 128)
                dd.append((x_sqb + y_sq[None, sl]) + dotm[:, sl])
            u01 = dd[1] < dd[0]
            p0 = jnp.where(u01, dd[1], dd[0])
            g0 = jnp.where(u01, g + 1, g)
            u23 = dd[3] < dd[2]
            p1 = jnp.where(u23, dd[3], dd[2])
            g1 = jnp.where(u23, g + 3, g + 2)
            u = p1 < p0
            pd = jnp.where(u, p1, p0)
            pg = jnp.where(u, g1, g0)
            upd = pd < runbest
            runbest = jnp.where(upd, pd, runbest)
            runstrip = jnp.where(upd, pg, runstrip)
    lane = lax.broadcasted_iota(jnp.int32, (TM, 128), 1)
    m = jnp.min(runbest, axis=1)
    absidx = runstrip * 128 + lane
    cand = jnp.where(runbest == m[:, None], absidx, V)
    idx_ref[...] = jnp.min(cand, axis=1).reshape(1, 1, TM)


def _argmin_indices(x2, bm):
    idx3 = pl.pallas_call(
        _argmin_body,
        grid=(GRID,),
        in_specs=[
            pl.BlockSpec((TM, C), lambda i: (i, 0)),
            pl.BlockSpec((C, V), lambda i: (0, 0)),
        ],
        out_specs=pl.BlockSpec((1, 1, TM), lambda i: (i, 0, 0)),
        out_shape=jax.ShapeDtypeStruct((GRID, 1, TM), jnp.int32),
    )(x2, bm)
    return idx3.reshape(N_TOK)


def _sc_gather(codebook, idx):
    info = plsc.get_sparse_core_info()
    nw = info.num_cores * info.num_subcores
    bpw = N_TOK // nw
    mesh = plsc.VectorSubcoreMesh(core_axis_name="c", subcore_axis_name="s")

    @functools.partial(
        pl.kernel, mesh=mesh,
        out_type=jax.ShapeDtypeStruct((N_TOK, C), jnp.float32),
        scratch_types=[
            pltpu.VMEM((bpw,), jnp.int32),
            pltpu.VMEM((bpw, C), jnp.float32),
            pltpu.SemaphoreType.DMA,
        ],
        compiler_params=pltpu.CompilerParams(use_tc_tiling_on_sc=False),
    )
    def gather_k(table_hbm, idx_hbm, out_hbm, idx_v, rows_v, sem):
        wid = lax.axis_index("s") * info.num_cores + lax.axis_index("c")
        base = wid * bpw
        pltpu.sync_copy(idx_hbm.at[pl.ds(base, bpw)], idx_v)
        pltpu.async_copy(table_hbm.at[idx_v], rows_v, sem).wait()
        pltpu.sync_copy(rows_v, out_hbm.at[pl.ds(base, bpw)])

    return gather_k(codebook, idx)


def kernel(x, codebook):
    x = x.astype(jnp.float32)
    x2 = x.reshape(N_TOK, C)
    bm = -2.0 * codebook.T
    idx = _argmin_indices(x2, bm)
    rows = _sc_gather(codebook, idx)
    vq_embed = rows.reshape(B, T, C)
    selected_indices = idx.reshape(B, T)
    vq_loss = jnp.zeros([], dtype=x.dtype)
    commit_loss = jnp.zeros([], dtype=x.dtype)
    return (vq_embed, selected_indices, vq_loss, commit_loss)


# speedup vs baseline: 1.0053x; 1.0053x over previous
import functools

import jax
import jax.numpy as jnp
from jax import lax
from jax.experimental import pallas as pl
from jax.experimental.pallas import tpu as pltpu
from jax.experimental.pallas import tpu_sc as plsc

B, T, C = 16, 1024, 32
V = 8192
N_TOK = B * T
TM = 4096
TN = 1024
GRID = N_TOK // TM
NCHUNK = V // TN


def _argmin_body(x_ref, bm_ref, idx_ref):
    xb = x_ref[...]
    x_sq = jnp.sum(xb * xb, axis=1, keepdims=True)
    x_sqb = jnp.broadcast_to(x_sq, (TM, 128))
    runbest = jnp.full((TM, 128), jnp.inf, dtype=jnp.float32)
    runstrip = jnp.zeros((TM, 128), dtype=jnp.int32)
    for j in range(NCHUNK):
        bmj = bm_ref[:, j * TN:(j + 1) * TN]
        y_sq = jnp.sum(bmj * bmj, axis=0) * 0.25
        dotm = lax.dot_general(
            xb, bmj, (((1,), (0,)), ((), ())),
            preferred_element_type=jnp.float32)
        for s in range(0, TN // 128, 2):
            g = j * (TN // 128) + s
            sl0 = slice(s * 128, (s + 1) * 128)
            sl1 = slice((s + 1) * 128, (s + 2) * 128)
            d0 = (x_sqb + y_sq[None, sl0]) + dotm[:, sl0]
            d1 = (x_sqb + y_sq[None, sl1]) + dotm[:, sl1]
            u = d1 < d0
            pd = jnp.where(u, d1, d0)
            pg = jnp.where(u, g + 1, g)
            upd = pd < runbest
            runbest = jnp.where(upd, pd, runbest)
            runstrip = jnp.where(upd, pg, runstrip)
    lane = lax.broadcasted_iota(jnp.int32, (TM, 128), 1)
    m = jnp.min(runbest, axis=1)
    absidx = runstrip * 128 + lane
    cand = jnp.where(runbest == m[:, None], absidx, V)
    idx_ref[...] = jnp.min(cand, axis=1).reshape(1, 1, TM)


def _argmin_indices(x2, bm):
    idx3 = pl.pallas_call(
        _argmin_body,
        grid=(GRID,),
        in_specs=[
            pl.BlockSpec((TM, C), lambda i: (i, 0)),
            pl.BlockSpec((C, V), lambda i: (0, 0)),
        ],
        out_specs=pl.BlockSpec((1, 1, TM), lambda i: (i, 0, 0)),
        out_shape=jax.ShapeDtypeStruct((GRID, 1, TM), jnp.int32),
    )(x2, bm)
    return idx3.reshape(N_TOK)


def _sc_gather(codebook, idx):
    info = plsc.get_sparse_core_info()
    nw = info.num_cores * info.num_subcores
    bpw = N_TOK // nw
    mesh = plsc.VectorSubcoreMesh(core_axis_name="c", subcore_axis_name="s")

    @functools.partial(
        pl.kernel, mesh=mesh,
        out_type=jax.ShapeDtypeStruct((N_TOK, C), jnp.float32),
        scratch_types=[
            pltpu.VMEM((bpw,), jnp.int32),
            pltpu.VMEM((bpw, C), jnp.float32),
            pltpu.SemaphoreType.DMA,
        ],
        compiler_params=pltpu.CompilerParams(use_tc_tiling_on_sc=False),
    )
    def gather_k(table_hbm, idx_hbm, out_hbm, idx_v, rows_v, sem):
        wid = lax.axis_index("s") * info.num_cores + lax.axis_index("c")
        base = wid * bpw
        pltpu.sync_copy(idx_hbm.at[pl.ds(base, bpw)], idx_v)
        pltpu.async_copy(table_hbm.at[idx_v], rows_v, sem).wait()
        pltpu.sync_copy(rows_v, out_hbm.at[pl.ds(base, bpw)])

    return gather_k(codebook, idx)


def kernel(x, codebook):
    x = x.astype(jnp.float32)
    x2 = x.reshape(N_TOK, C)
    bm = -2.0 * codebook.T
    idx = _argmin_indices(x2, bm)
    rows = _sc_gather(codebook, idx)
    vq_embed = rows.reshape(B, T, C)
    selected_indices = idx.reshape(B, T)
    vq_loss = jnp.zeros([], dtype=x.dtype)
    commit_loss = jnp.zeros([], dtype=x.dtype)
    return (vq_embed, selected_indices, vq_loss, commit_loss)
